# baseline (speedup 1.0000x reference)
import functools

import jax
import jax.numpy as jnp
from jax import lax
from jax.experimental import pallas as pl
from jax.experimental.pallas import tpu as pltpu
from jax.experimental.pallas import tpu_sc as plsc

_N = 10000
_E = 320000
_HID = 64
_NG = 16
_NC = 2
_NS = 16
_NP = 10240
_RPT = _NP // _NS
_BLK = 128
_NBLK = _E // _BLK
_NBT = 78
_RING = 6
_RG = 10
_RB = _N // _RG

_mesh = plsc.VectorSubcoreMesh(core_axis_name="c", subcore_axis_name="s")
_sc_params = pltpu.CompilerParams(use_tc_tiling_on_sc=False)


@functools.partial(
    pl.kernel,
    out_type=jax.ShapeDtypeStruct((_NC * _NP,), jnp.float32),
    mesh=_mesh,
    compiler_params=_sc_params,
    scratch_types=[
        pltpu.VMEM((_NBT + 1, _BLK), jnp.int32),
        pltpu.VMEM((_BLK,), jnp.float32),
        pltpu.VMEM((_RPT,), jnp.float32),
        pltpu.VMEM_SHARED((_NP,), jnp.float32),
        pltpu.SemaphoreType.DMA,
    ],
)
def _sc_degree(dst_hbm, out_hbm, idx_v, ones_v, zero_v, acc_sh, sem):
    c = lax.axis_index("c")
    s = lax.axis_index("s")
    wid = c * _NS + s

    @pl.loop(0, _BLK, step=16)
    def _(i):
        ones_v[pl.ds(i, 16)] = jnp.ones((16,), jnp.float32)

    @pl.loop(0, _RPT, step=16)
    def _(i):
        zero_v[pl.ds(i, 16)] = jnp.zeros((16,), jnp.float32)

    pltpu.sync_copy(zero_v, acc_sh.at[pl.ds(s * _RPT, _RPT)])
    pltpu.sync_copy(dst_hbm.at[pl.ds(wid * _NBT, _NBT)],
                    idx_v.at[pl.ds(0, _NBT)])

    @pl.when(wid < _NBLK - _NC * _NS * _NBT)
    def _():
        pltpu.sync_copy(dst_hbm.at[pl.ds(_NC * _NS * _NBT + wid, 1)],
                        idx_v.at[pl.ds(_NBT, 1)])

    plsc.subcore_barrier()

    @pl.loop(0, _NBT, step=13)
    def _(k):
        for j in range(13):
            pltpu.async_copy(ones_v, acc_sh.at[idx_v.at[k + j]], sem,
                             add=True)
        for j in range(13):
            pltpu.make_async_copy(ones_v, acc_sh.at[idx_v.at[k + j]],
                                  sem).wait()

    @pl.when(wid < _NBLK - _NC * _NS * _NBT)
    def _():
        pltpu.sync_copy(ones_v, acc_sh.at[idx_v.at[_NBT]], add=True)

    plsc.subcore_barrier()
    pltpu.sync_copy(acc_sh.at[pl.ds(s * _RPT, _RPT)], zero_v)
    pltpu.sync_copy(zero_v, out_hbm.at[pl.ds(c * _NP + s * _RPT, _RPT)])


@functools.partial(
    pl.kernel,
    out_type=jax.ShapeDtypeStruct((_NC, _NP, _HID), jnp.float32),
    mesh=_mesh,
    compiler_params=_sc_params,
    scratch_types=[
        pltpu.VMEM((_NBT + 1, _BLK), jnp.int32),
        pltpu.VMEM((_NBT + 1, _BLK), jnp.int32),
        pltpu.VMEM((_RING, _BLK, _HID), jnp.float32),
        pltpu.VMEM_SHARED((_NP, _HID), jnp.float32),
        pltpu.SemaphoreType.DMA,
    ] + [pltpu.SemaphoreType.DMA] * (2 * _RING),
)
def _sc_aggregate(table_hbm, src_hbm, dst_hbm, out_hbm,
                  isrc_v, idst_v, rows_v, acc_sh, sem, *slot_sems):
    c = lax.axis_index("c")
    s = lax.axis_index("s")
    wid = c * _NS + s
    extra = wid < _NBLK - _NC * _NS * _NBT
    sg = slot_sems[:_RING]
    ss = slot_sems[_RING:]

    def fire_gather(blk, b):
        pltpu.async_copy(table_hbm.at[isrc_v.at[blk]], rows_v.at[b], sg[b])

    def wait_gather(blk, b):
        pltpu.make_async_copy(table_hbm.at[isrc_v.at[blk]], rows_v.at[b],
                              sg[b]).wait()

    def fire_scatter(blk, b):
        pltpu.async_copy(rows_v.at[b], acc_sh.at[idst_v.at[blk]], ss[b],
                         add=True)

    def wait_scatter(blk, b):
        pltpu.make_async_copy(rows_v.at[b], acc_sh.at[idst_v.at[blk]],
                              ss[b]).wait()

    @pl.loop(0, _BLK)
    def _(r):
        @pl.loop(0, _HID, step=16)
        def _(j):
            rows_v[0, r, pl.ds(j, 16)] = jnp.zeros((16,), jnp.float32)

    @pl.loop(0, _RPT, step=_BLK)
    def _(r):
        pltpu.sync_copy(rows_v.at[0], acc_sh.at[pl.ds(s * _RPT + r, _BLK)])

    pltpu.sync_copy(src_hbm.at[pl.ds(wid * _NBT, _NBT)],
                    isrc_v.at[pl.ds(0, _NBT)])
    pltpu.sync_copy(dst_hbm.at[pl.ds(wid * _NBT, _NBT)],
                    idst_v.at[pl.ds(0, _NBT)])

    @pl.when(extra)
    def _():
        pltpu.sync_copy(src_hbm.at[pl.ds(_NC * _NS * _NBT + wid, 1)],
                        isrc_v.at[pl.ds(_NBT, 1)])
        pltpu.sync_copy(dst_hbm.at[pl.ds(_NC * _NS * _NBT + wid, 1)],
                        idst_v.at[pl.ds(_NBT, 1)])

    plsc.subcore_barrier()

    for b in range(_RING):
        fire_gather(b, b)
    for b in range(_RING):
        wait_gather(b, b)
        fire_scatter(b, b)

    @pl.loop(_RING, _NBT, step=_RING)
    def _(g):
        for b in range(_RING):
            wait_scatter(g - _RING + b, b)
            fire_gather(g + b, b)
        for b in range(_RING):
            wait_gather(g + b, b)
            fire_scatter(g + b, b)

    for b in range(_RING):
        wait_scatter(_NBT - _RING + b, b)

    @pl.when(extra)
    def _():
        pltpu.sync_copy(table_hbm.at[isrc_v.at[_NBT]], rows_v.at[0])
        pltpu.sync_copy(rows_v.at[0], acc_sh.at[idst_v.at[_NBT]], add=True)

    plsc.subcore_barrier()
    @pl.loop(0, _RPT, step=_BLK)
    def _(r):
        pltpu.sync_copy(acc_sh.at[pl.ds(s * _RPT + r, _BLK)], rows_v.at[0])
        pltpu.sync_copy(rows_v.at[0], out_hbm.at[c, pl.ds(s * _RPT + r, _BLK)])


def _dot(a, b):
    return jnp.dot(a, b, preferred_element_type=jnp.float32,
                   precision=lax.Precision.HIGHEST)


def _dinv_block(degp_ref, i):
    del i
    deg = degp_ref[0, 0, :] + degp_ref[0, 1, :] + 1.0
    return lax.rsqrt(deg)[:, None]


_degp_spec = pl.BlockSpec((1, 2, _RB), lambda i: (i, 0, 0))


def _tc_first_body(x_ref, w_ref, degp_ref, o_ref):
    i = pl.program_id(0)
    o_ref[...] = _dinv_block(degp_ref, i) * _dot(x_ref[...], w_ref[...])


_tc_first = pl.pallas_call(
    _tc_first_body,
    grid=(_RG,),
    in_specs=[
        pl.BlockSpec((_RB, 128), lambda i: (i, 0)),
        pl.BlockSpec((128, _HID), lambda i: (0, 0)),
        _degp_spec,
    ],
    out_specs=pl.BlockSpec((_RB, _HID), lambda i: (i, 0)),
    out_shape=jax.ShapeDtypeStruct((_N, _HID), jnp.float32),
)


def _tc_mid_body(p_ref, g_ref, degp_ref, b_ref, w_ref, o_ref):
    i = pl.program_id(0)
    dinv = _dinv_block(degp_ref, i)
    h = p_ref[0] + p_ref[1] + g_ref[...]
    h = jnp.maximum(dinv * h + b_ref[...][None, :], 0.0)
    o_ref[...] = dinv * _dot(h, w_ref[...])


_tc_mid = pl.pallas_call(
    _tc_mid_body,
    grid=(_RG,),
    in_specs=[
        pl.BlockSpec((2, _RB, _HID), lambda i: (0, i, 0)),
        pl.BlockSpec((_RB, _HID), lambda i: (i, 0)),
        _degp_spec,
        pl.BlockSpec((_HID,), lambda i: (0,)),
        pl.BlockSpec((_HID, _HID), lambda i: (0, 0)),
    ],
    out_specs=pl.BlockSpec((_RB, _HID), lambda i: (i, 0)),
    out_shape=jax.ShapeDtypeStruct((_N, _HID), jnp.float32),
)


def _tc_final_body(p_ref, g_ref, degp_ref, b_ref, batch_ref, lw_ref, lb_ref,
                   o_ref, sacc, cacc):
    i = pl.program_id(0)

    @pl.when(i == 0)
    def _():
        sacc[...] = jnp.zeros((_NG, _HID), jnp.float32)
        cacc[...] = jnp.zeros((_NG, 1), jnp.float32)

    dinv = _dinv_block(degp_ref, i)
    h = p_ref[0] + p_ref[1] + g_ref[...]
    h = jnp.maximum(dinv * h + b_ref[...][None, :], 0.0)
    labels = lax.broadcasted_iota(jnp.int32, (1, _NG), 1)
    onehot = (batch_ref[...] == labels).astype(jnp.float32)
    sacc[...] += lax.dot_general(onehot, h, (((0,), (0,)), ((), ())),
                                 preferred_element_type=jnp.float32,
                                 precision=lax.Precision.HIGHEST)
    cacc[...] += jnp.sum(onehot, axis=0)[:, None]

    @pl.when(i == _RG - 1)
    def _():
        pooled = sacc[...] / jnp.maximum(cacc[...], 1.0)
        o_ref[...] = _dot(pooled, lw_ref[...]) + lb_ref[...][None, :]


_tc_final = pl.pallas_call(
    _tc_final_body,
    grid=(_RG,),
    in_specs=[
        pl.BlockSpec((2, _RB, _HID), lambda i: (0, i, 0)),
        pl.BlockSpec((_RB, _HID), lambda i: (i, 0)),
        _degp_spec,
        pl.BlockSpec((_HID,), lambda i: (0,)),
        pl.BlockSpec((_RB, 1), lambda i: (i, 0)),
        pl.BlockSpec((_HID, 2), lambda i: (0, 0)),
        pl.BlockSpec((2,), lambda i: (0,)),
    ],
    out_specs=pl.BlockSpec((_NG, 2), lambda i: (0, 0)),
    out_shape=jax.ShapeDtypeStruct((_NG, 2), jnp.float32),
    scratch_shapes=[
        pltpu.VMEM((_NG, _HID), jnp.float32),
        pltpu.VMEM((_NG, 1), jnp.float32),
    ],
)


def kernel(x, edge_index, batch, W1, b1, W2, b2, W3, b3, lin_W, lin_b):
    ei = edge_index.astype(jnp.int32).reshape(2, _NBLK, _BLK)
    src2 = ei[0]
    dst2 = ei[1]
    batch2 = batch.astype(jnp.int32).reshape(_N, 1)

    degp = _sc_degree(dst2)
    degp2 = (degp.reshape(2, _NP)[:, :_N]
             .reshape(2, _RG, _RB).transpose(1, 0, 2))
    g1 = _tc_first(x, W1, degp2)
    p1 = _sc_aggregate(g1, src2, dst2)
    g2 = _tc_mid(p1, g1, degp2, b1, W2)
    p2 = _sc_aggregate(g2, src2, dst2)
    g3 = _tc_mid(p2, g2, degp2, b2, W3)
    p3 = _sc_aggregate(g3, src2, dst2)
    return _tc_final(p3, g3, degp2, b3, batch2, lin_W, lin_b)


# speedup vs baseline: 1.0357x; 1.0357x over previous
import functools

import jax
import jax.numpy as jnp
from jax import lax
from jax.experimental import pallas as pl
from jax.experimental.pallas import tpu as pltpu
from jax.experimental.pallas import tpu_sc as plsc

_N = 10000
_E = 320000
_HID = 64
_NG = 16
_NC = 2
_NS = 16
_NP = 10240
_RPT = _NP // _NS
_BLK = 128
_NBLK = _E // _BLK
_NBT = 78
_RING = 6
_RG = 10
_RB = _N // _RG

_mesh = plsc.VectorSubcoreMesh(core_axis_name="c", subcore_axis_name="s")
_sc_params = pltpu.CompilerParams(use_tc_tiling_on_sc=False)


@functools.partial(
    pl.kernel,
    out_type=jax.ShapeDtypeStruct((_NC * _NP,), jnp.float32),
    mesh=_mesh,
    compiler_params=_sc_params,
    scratch_types=[
        pltpu.VMEM((_NBT + 1, _BLK), jnp.int32),
        pltpu.VMEM((_BLK,), jnp.float32),
        pltpu.VMEM((_RPT,), jnp.float32),
        pltpu.VMEM_SHARED((_NP,), jnp.float32),
        pltpu.SemaphoreType.DMA,
    ],
)
def _sc_degree(dst_hbm, out_hbm, idx_v, ones_v, zero_v, acc_sh, sem):
    c = lax.axis_index("c")
    s = lax.axis_index("s")
    wid = c * _NS + s

    @pl.loop(0, _BLK, step=16)
    def _(i):
        ones_v[pl.ds(i, 16)] = jnp.ones((16,), jnp.float32)

    @pl.loop(0, _RPT, step=16)
    def _(i):
        zero_v[pl.ds(i, 16)] = jnp.zeros((16,), jnp.float32)

    pltpu.sync_copy(zero_v, acc_sh.at[pl.ds(s * _RPT, _RPT)])
    pltpu.sync_copy(dst_hbm.at[pl.ds(wid * _NBT, _NBT)],
                    idx_v.at[pl.ds(0, _NBT)])

    @pl.when(wid < _NBLK - _NC * _NS * _NBT)
    def _():
        pltpu.sync_copy(dst_hbm.at[pl.ds(_NC * _NS * _NBT + wid, 1)],
                        idx_v.at[pl.ds(_NBT, 1)])

    plsc.subcore_barrier()

    @pl.loop(0, _NBT, step=13)
    def _(k):
        for j in range(13):
            pltpu.async_copy(ones_v, acc_sh.at[idx_v.at[k + j]], sem,
                             add=True)
        for j in range(13):
            pltpu.make_async_copy(ones_v, acc_sh.at[idx_v.at[k + j]],
                                  sem).wait()

    @pl.when(wid < _NBLK - _NC * _NS * _NBT)
    def _():
        pltpu.sync_copy(ones_v, acc_sh.at[idx_v.at[_NBT]], add=True)

    plsc.subcore_barrier()
    pltpu.sync_copy(acc_sh.at[pl.ds(s * _RPT, _RPT)], zero_v)
    pltpu.sync_copy(zero_v, out_hbm.at[pl.ds(c * _NP + s * _RPT, _RPT)])


@functools.partial(
    pl.kernel,
    out_type=jax.ShapeDtypeStruct((_NC, _NP, _HID), jnp.float32),
    mesh=_mesh,
    compiler_params=_sc_params,
    scratch_types=[
        pltpu.VMEM((_NBT + 1, _BLK), jnp.int32),
        pltpu.VMEM((_NBT + 1, _BLK), jnp.int32),
        pltpu.VMEM((_RING, _BLK, _HID), jnp.float32),
        pltpu.VMEM_SHARED((_NP, _HID), jnp.float32),
        pltpu.SemaphoreType.DMA,
    ] + [pltpu.SemaphoreType.DMA] * (2 * _RING),
)
def _sc_aggregate(table_hbm, src_hbm, dst_hbm, out_hbm,
                  isrc_v, idst_v, rows_v, acc_sh, sem, *slot_sems):
    c = lax.axis_index("c")
    s = lax.axis_index("s")
    wid = c * _NS + s
    extra = wid < _NBLK - _NC * _NS * _NBT
    sg = slot_sems[:_RING]
    ss = slot_sems[_RING:]

    def fire_gather(blk, b):
        pltpu.async_copy(table_hbm.at[isrc_v.at[blk]], rows_v.at[b], sg[b])

    def wait_gather(blk, b):
        pltpu.make_async_copy(table_hbm.at[isrc_v.at[blk]], rows_v.at[b],
                              sg[b]).wait()

    def fire_scatter(blk, b):
        pltpu.async_copy(rows_v.at[b], acc_sh.at[idst_v.at[blk]], ss[b],
                         add=True)

    def wait_scatter(blk, b):
        pltpu.make_async_copy(rows_v.at[b], acc_sh.at[idst_v.at[blk]],
                              ss[b]).wait()

    @pl.loop(0, _BLK)
    def _(r):
        @pl.loop(0, _HID, step=16)
        def _(j):
            rows_v[0, r, pl.ds(j, 16)] = jnp.zeros((16,), jnp.float32)

    @pl.loop(0, _RPT, step=_BLK)
    def _(r):
        pltpu.sync_copy(rows_v.at[0], acc_sh.at[pl.ds(s * _RPT + r, _BLK)])

    pltpu.sync_copy(src_hbm.at[pl.ds(wid * _NBT, _NBT)],
                    isrc_v.at[pl.ds(0, _NBT)])
    pltpu.sync_copy(dst_hbm.at[pl.ds(wid * _NBT, _NBT)],
                    idst_v.at[pl.ds(0, _NBT)])

    @pl.when(extra)
    def _():
        pltpu.sync_copy(src_hbm.at[pl.ds(_NC * _NS * _NBT + wid, 1)],
                        isrc_v.at[pl.ds(_NBT, 1)])
        pltpu.sync_copy(dst_hbm.at[pl.ds(_NC * _NS * _NBT + wid, 1)],
                        idst_v.at[pl.ds(_NBT, 1)])

    plsc.subcore_barrier()

    for b in range(_RING):
        fire_gather(b, b)
    for b in range(_RING):
        wait_gather(b, b)
        fire_scatter(b, b)

    @pl.loop(_RING, _NBT, step=_RING)
    def _(g):
        for b in range(_RING):
            wait_scatter(g - _RING + b, b)
            fire_gather(g + b, b)
        for b in range(_RING):
            wait_gather(g + b, b)
            fire_scatter(g + b, b)

    for b in range(_RING):
        wait_scatter(_NBT - _RING + b, b)

    @pl.when(extra)
    def _():
        pltpu.sync_copy(table_hbm.at[isrc_v.at[_NBT]], rows_v.at[0])
        pltpu.sync_copy(rows_v.at[0], acc_sh.at[idst_v.at[_NBT]], add=True)

    plsc.subcore_barrier()
    @pl.loop(0, _RPT, step=_BLK)
    def _(r):
        pltpu.sync_copy(acc_sh.at[pl.ds(s * _RPT + r, _BLK)], rows_v.at[0])
        pltpu.sync_copy(rows_v.at[0], out_hbm.at[c, pl.ds(s * _RPT + r, _BLK)])


def _dot(a, b):
    return jnp.dot(a, b, preferred_element_type=jnp.float32,
                   precision=lax.Precision.HIGHEST)


def _dinv_of(degp_ref):
    deg = degp_ref[:_N] + degp_ref[_NP:_NP + _N] + 1.0
    return lax.rsqrt(deg)[:, None]


def _tc_first_body(x_ref, w_ref, degp_ref, o_ref):
    o_ref[...] = _dinv_of(degp_ref) * _dot(x_ref[...], w_ref[...])


_tc_first = pl.pallas_call(
    _tc_first_body,
    out_shape=jax.ShapeDtypeStruct((_N, _HID), jnp.float32),
)


def _tc_mid_body(p_ref, g_ref, degp_ref, b_ref, w_ref, o_ref):
    dinv = _dinv_of(degp_ref)
    h = p_ref[0, :_N, :] + p_ref[1, :_N, :] + g_ref[...]
    h = jnp.maximum(dinv * h + b_ref[...][None, :], 0.0)
    o_ref[...] = dinv * _dot(h, w_ref[...])


_tc_mid = pl.pallas_call(
    _tc_mid_body,
    out_shape=jax.ShapeDtypeStruct((_N, _HID), jnp.float32),
)


def _tc_final_body(p_ref, g_ref, degp_ref, b_ref, batch_ref, lw_ref, lb_ref,
                   o_ref):
    dinv = _dinv_of(degp_ref)
    h = p_ref[0, :_N, :] + p_ref[1, :_N, :] + g_ref[...]
    h = jnp.maximum(dinv * h + b_ref[...][None, :], 0.0)
    labels = lax.broadcasted_iota(jnp.int32, (1, _NG), 1)
    onehot = (batch_ref[...] == labels).astype(jnp.float32)
    sums = lax.dot_general(onehot, h, (((0,), (0,)), ((), ())),
                           preferred_element_type=jnp.float32,
                           precision=lax.Precision.HIGHEST)
    counts = jnp.sum(onehot, axis=0)[:, None]
    pooled = sums / jnp.maximum(counts, 1.0)
    o_ref[...] = _dot(pooled, lw_ref[...]) + lb_ref[...][None, :]


_tc_final = pl.pallas_call(
    _tc_final_body,
    out_shape=jax.ShapeDtypeStruct((_NG, 2), jnp.float32),
)


def kernel(x, edge_index, batch, W1, b1, W2, b2, W3, b3, lin_W, lin_b):
    ei = edge_index.astype(jnp.int32).reshape(2, _NBLK, _BLK)
    src2 = ei[0]
    dst2 = ei[1]
    batch2 = batch.astype(jnp.int32).reshape(_N, 1)

    degp = _sc_degree(dst2)
    g1 = _tc_first(x, W1, degp)
    p1 = _sc_aggregate(g1, src2, dst2)
    g2 = _tc_mid(p1, g1, degp, b1, W2)
    p2 = _sc_aggregate(g2, src2, dst2)
    g3 = _tc_mid(p2, g2, degp, b2, W3)
    p3 = _sc_aggregate(g3, src2, dst2)
    return _tc_final(p3, g3, degp, b3, batch2, lin_W, lin_b)


# speedup vs baseline: 1.2118x; 1.1700x over previous
import functools

import jax
import jax.numpy as jnp
from jax import lax
from jax.experimental import pallas as pl
from jax.experimental.pallas import tpu as pltpu
from jax.experimental.pallas import tpu_sc as plsc

_N = 10000
_E = 320000
_HID = 64
_NG = 16
_NC = 2
_NS = 16
_NP = 10240
_RPT = _NP // _NS
_BLK = 128
_NBLK = _E // _BLK
_NBT = 78
_RING = 6
_RG = 10
_RB = _N // _RG

_mesh = plsc.VectorSubcoreMesh(core_axis_name="c", subcore_axis_name="s")
_sc_params = pltpu.CompilerParams(use_tc_tiling_on_sc=False)


@functools.partial(
    pl.kernel,
    out_type=jax.ShapeDtypeStruct((_NC * _NP,), jnp.float32),
    mesh=_mesh,
    compiler_params=_sc_params,
    scratch_types=[
        pltpu.VMEM((_NBT + 1, _BLK), jnp.int32),
        pltpu.VMEM((_BLK,), jnp.float32),
        pltpu.VMEM((_RPT,), jnp.float32),
        pltpu.VMEM_SHARED((_NP,), jnp.float32),
        pltpu.SemaphoreType.DMA,
    ],
)
def _sc_degree(dst_hbm, out_hbm, idx_v, ones_v, zero_v, acc_sh, sem):
    c = lax.axis_index("c")
    s = lax.axis_index("s")
    wid = c * _NS + s

    @pl.loop(0, _BLK, step=16)
    def _(i):
        ones_v[pl.ds(i, 16)] = jnp.ones((16,), jnp.float32)

    @pl.loop(0, _RPT, step=16)
    def _(i):
        zero_v[pl.ds(i, 16)] = jnp.zeros((16,), jnp.float32)

    pltpu.sync_copy(zero_v, acc_sh.at[pl.ds(s * _RPT, _RPT)])
    pltpu.sync_copy(dst_hbm.at[pl.ds(wid * _NBT, _NBT)],
                    idx_v.at[pl.ds(0, _NBT)])

    @pl.when(wid < _NBLK - _NC * _NS * _NBT)
    def _():
        pltpu.sync_copy(dst_hbm.at[pl.ds(_NC * _NS * _NBT + wid, 1)],
                        idx_v.at[pl.ds(_NBT, 1)])

    plsc.subcore_barrier()

    @pl.loop(0, _NBT, step=13)
    def _(k):
        for j in range(13):
            pltpu.async_copy(ones_v, acc_sh.at[idx_v.at[k + j]], sem,
                             add=True)
        for j in range(13):
            pltpu.make_async_copy(ones_v, acc_sh.at[idx_v.at[k + j]],
                                  sem).wait()

    @pl.when(wid < _NBLK - _NC * _NS * _NBT)
    def _():
        pltpu.sync_copy(ones_v, acc_sh.at[idx_v.at[_NBT]], add=True)

    plsc.subcore_barrier()
    pltpu.sync_copy(acc_sh.at[pl.ds(s * _RPT, _RPT)], zero_v)
    pltpu.sync_copy(zero_v, out_hbm.at[pl.ds(c * _NP + s * _RPT, _RPT)])


@functools.partial(
    pl.kernel,
    out_type=jax.ShapeDtypeStruct((_NC, _NP, _HID), jnp.float32),
    mesh=_mesh,
    compiler_params=_sc_params,
    scratch_types=[
        pltpu.VMEM((_NBT + 1, _BLK), jnp.int32),
        pltpu.VMEM((_NBT + 1, _BLK), jnp.int32),
        pltpu.VMEM((_RING, _BLK, _HID), jnp.float32),
        pltpu.VMEM_SHARED((_NP, _HID), jnp.float32),
        pltpu.SemaphoreType.DMA,
    ] + [pltpu.SemaphoreType.DMA] * (2 * _RING),
)
def _sc_aggregate(table_hbm, src_hbm, dst_hbm, out_hbm,
                  isrc_v, idst_v, rows_v, acc_sh, sem, *slot_sems):
    c = lax.axis_index("c")
    s = lax.axis_index("s")
    wid = c * _NS + s
    extra = wid < _NBLK - _NC * _NS * _NBT
    sg = slot_sems[:_RING]
    ss = slot_sems[_RING:]

    def fire_gather(blk, b):
        pltpu.async_copy(table_hbm.at[isrc_v.at[blk]], rows_v.at[b], sg[b])

    def wait_gather(blk, b):
        pltpu.make_async_copy(table_hbm.at[isrc_v.at[blk]], rows_v.at[b],
                              sg[b]).wait()

    def fire_scatter(blk, b):
        pltpu.async_copy(rows_v.at[b], acc_sh.at[idst_v.at[blk]], ss[b],
                         add=True)

    def wait_scatter(blk, b):
        pltpu.make_async_copy(rows_v.at[b], acc_sh.at[idst_v.at[blk]],
                              ss[b]).wait()

    @pl.loop(0, _BLK)
    def _(r):
        @pl.loop(0, _HID, step=16)
        def _(j):
            rows_v[0, r, pl.ds(j, 16)] = jnp.zeros((16,), jnp.float32)

    @pl.loop(0, _RPT, step=_BLK)
    def _(r):
        pltpu.sync_copy(rows_v.at[0], acc_sh.at[pl.ds(s * _RPT + r, _BLK)])

    pltpu.sync_copy(src_hbm.at[pl.ds(wid * _NBT, _NBT)],
                    isrc_v.at[pl.ds(0, _NBT)])
    pltpu.sync_copy(dst_hbm.at[pl.ds(wid * _NBT, _NBT)],
                    idst_v.at[pl.ds(0, _NBT)])

    @pl.when(extra)
    def _():
        pltpu.sync_copy(src_hbm.at[pl.ds(_NC * _NS * _NBT + wid, 1)],
                        isrc_v.at[pl.ds(_NBT, 1)])
        pltpu.sync_copy(dst_hbm.at[pl.ds(_NC * _NS * _NBT + wid, 1)],
                        idst_v.at[pl.ds(_NBT, 1)])

    plsc.subcore_barrier()

    for b in range(_RING):
        fire_gather(b, b)
    for b in range(_RING):
        wait_gather(b, b)
        fire_scatter(b, b)

    @pl.loop(_RING, _NBT, step=_RING)
    def _(g):
        for b in range(_RING):
            wait_scatter(g - _RING + b, b)
            fire_gather(g + b, b)
        for b in range(_RING):
            wait_gather(g + b, b)
            fire_scatter(g + b, b)

    for b in range(_RING):
        wait_scatter(_NBT - _RING + b, b)

    @pl.when(extra)
    def _():
        pltpu.sync_copy(table_hbm.at[isrc_v.at[_NBT]], rows_v.at[0])
        pltpu.sync_copy(rows_v.at[0], acc_sh.at[idst_v.at[_NBT]], add=True)

    plsc.subcore_barrier()
    @pl.loop(0, _RPT, step=_BLK)
    def _(r):
        pltpu.sync_copy(acc_sh.at[pl.ds(s * _RPT + r, _BLK)], rows_v.at[0])
        pltpu.sync_copy(rows_v.at[0], out_hbm.at[c, pl.ds(s * _RPT + r, _BLK)])


def _dot(a, b):
    return jnp.dot(a, b, preferred_element_type=jnp.float32,
                   precision=lax.Precision.HIGHEST)


_NF = _NP // 2
_NFR = _N // 2


def _tc_first_body(x_ref, w_ref, degp_ref, o_ref, od_ref):
    deg = degp_ref[:_N] + degp_ref[_NP:_NP + _N] + 1.0
    dinv = lax.rsqrt(deg)[:, None]
    g = dinv * _dot(x_ref[...], w_ref[...])
    zpad = jnp.zeros((_NP - _N, _HID), jnp.float32)
    o_ref[...] = jnp.concatenate(
        [g[:_NF, :], jnp.concatenate([g[_NF:, :], zpad], 0)], 1)
    db = jnp.broadcast_to(dinv, (_N, _HID))
    od_ref[...] = jnp.concatenate(
        [db[:_NF, :],
         jnp.concatenate([db[_NF:, :], zpad + 1.0], 0)], 1)


_tc_first = pl.pallas_call(
    _tc_first_body,
    out_shape=[jax.ShapeDtypeStruct((_NF, 128), jnp.float32),
               jax.ShapeDtypeStruct((_NF, 128), jnp.float32)],
)


def _blockdiag(w):
    z = jnp.zeros((_HID, _HID), jnp.float32)
    return jnp.concatenate([jnp.concatenate([w, z], 1),
                            jnp.concatenate([z, w], 1)], 0)


def _tc_mid_body(p_ref, g_ref, dinv_ref, b_ref, w_ref, o_ref):
    dinv = dinv_ref[...]
    b2 = jnp.concatenate([b_ref[...], b_ref[...]])[None, :]
    h = p_ref[0] + p_ref[1] + g_ref[...]
    h = jnp.maximum(dinv * h + b2, 0.0)
    o_ref[...] = dinv * _dot(h, _blockdiag(w_ref[...]))


_tc_mid = pl.pallas_call(
    _tc_mid_body,
    out_shape=jax.ShapeDtypeStruct((_NF, 128), jnp.float32),
)


def _tc_final_body(p_ref, g_ref, dinv_ref, b_ref, batch_ref, lw_ref, lb_ref,
                   o_ref):
    b2 = jnp.concatenate([b_ref[...], b_ref[...]])[None, :]
    h = p_ref[0] + p_ref[1] + g_ref[...]
    h = jnp.maximum(dinv_ref[...] * h + b2, 0.0)
    labels = lax.broadcasted_iota(jnp.int32, (1, _NG), 1)
    oh_e = (batch_ref[:, 0][:, None] == labels).astype(jnp.float32)
    oh_o = (batch_ref[:, 1][:, None] == labels).astype(jnp.float32)
    dn = (((0,), (0,)), ((), ()))
    sums = (lax.dot_general(oh_e, h[:, :_HID], dn,
                            preferred_element_type=jnp.float32,
                            precision=lax.Precision.HIGHEST)
            + lax.dot_general(oh_o, h[:, _HID:], dn,
                              preferred_element_type=jnp.float32,
                              precision=lax.Precision.HIGHEST))
    counts = (jnp.sum(oh_e, axis=0) + jnp.sum(oh_o, axis=0))[:, None]
    pooled = sums / jnp.maximum(counts, 1.0)
    o_ref[...] = _dot(pooled, lw_ref[...]) + lb_ref[...][None, :]


_tc_final = pl.pallas_call(
    _tc_final_body,
    out_shape=jax.ShapeDtypeStruct((_NG, 2), jnp.float32),
)


def kernel(x, edge_index, batch, W1, b1, W2, b2, W3, b3, lin_W, lin_b):
    ei = edge_index.astype(jnp.int32)
    remap = lambda v: jnp.where(v < _NF, 2 * v, 2 * v - (_NP - 1))
    src2r = remap(ei[0]).reshape(_NBLK, _BLK)
    dst2r = remap(ei[1]).reshape(_NBLK, _BLK)
    dst2 = ei[1].reshape(_NBLK, _BLK)
    batchp = jnp.concatenate([batch.astype(jnp.int32),
                              jnp.full((_NP - _N,), -1, jnp.int32)])
    batch2 = batchp.reshape(2, _NF).T

    def unfold(t):
        return t.reshape(_NP, _HID)

    def fold(p):
        return p.reshape(2, _NF, 128)

    degp = _sc_degree(dst2)
    g1, dinvf = _tc_first(x, W1, degp)
    p1 = _sc_aggregate(unfold(g1), src2r, dst2r)
    g2 = _tc_mid(fold(p1), g1, dinvf, b1, W2)
    p2 = _sc_aggregate(unfold(g2), src2r, dst2r)
    g3 = _tc_mid(fold(p2), g2, dinvf, b2, W3)
    p3 = _sc_aggregate(unfold(g3), src2r, dst2r)
    return _tc_final(fold(p3), g3, dinvf, b3, batch2, lin_W, lin_b)
